# baseline (speedup 1.0000x reference)
import functools

import jax
import jax.numpy as jnp
from jax import lax
from jax.experimental import pallas as pl
from jax.experimental.pallas import tpu as pltpu
from jax.experimental.pallas import tpu_sc as plsc

_NC = 2
_NS = 16
_NW = _NC * _NS
_BN = 128
_C = 8
_ROWS = 2048
_ED = 168


def _ssp_sc_kernel(n: int, chunk: int, n_chunk: int):
  blocks_per_w = n_chunk // _BN // _NW
  col0 = chunk * n_chunk
  mesh = plsc.VectorSubcoreMesh(core_axis_name="c", subcore_axis_name="s")

  @functools.partial(
      pl.kernel,
      out_type=jax.ShapeDtypeStruct((_ED, n_chunk), jnp.float32),
      mesh=mesh,
      scratch_types=[
          pltpu.VMEM((256, _BN), jnp.float32),
          pltpu.VMEM((256, _BN), jnp.float32),
          pltpu.VMEM((_ED, _BN), jnp.float32),
          pltpu.SemaphoreType.DMA,
          pltpu.SemaphoreType.DMA,
      ],
      compiler_params=pltpu.CompilerParams(needs_layout_passes=False),
  )
  def ssp(pt_hbm, emb_hbm, buf0, buf1, ebuf, sem0, sem1):
    wid = lax.axis_index("s") * _NC + lax.axis_index("c")
    bufs = (buf0, buf1)
    sems = (sem0, sem1)

    def block_body(k, carry):
      n0 = (wid * blocks_per_w + k) * _BN

      def start(c, slot):
        return pltpu.async_copy(
            pt_hbm.at[pl.ds(c * 256, 256), pl.ds(col0 + n0, _BN)], bufs[slot],
            sems[slot])

      desc = start(0, 0)
      for c in range(_C):
        desc.wait()
        if c + 1 < _C:
          desc = start(c + 1, (c + 1) % 2)
        buf = bufs[c % 2]

        def gj_body(gj, carry2):
          g = gj // 4
          j4 = gj % 4
          accs = []
          for v in range(_BN // 16):
            acc = None
            for dy in range(4):
              for dx in range(4):
                val = buf[(g * 4 + dy) * 16 + j4 * 4 + dx, pl.ds(v * 16, 16)]
                acc = val if acc is None else jnp.maximum(acc, val)
            accs.append(acc)
          for v in range(_BN // 16):
            ebuf[40 + c * 16 + gj, pl.ds(v * 16, 16)] = accs[v]
          return carry2

        lax.fori_loop(0, 16, gj_body, 0, unroll=False)

        for i in range(2):
          for j in range(2):
            for v in range(_BN // 16):
              acc = None
              for di in range(2):
                for dj in range(2):
                  val = ebuf[40 + c * 16 + (2 * i + di) * 4 + (2 * j + dj),
                             pl.ds(v * 16, 16)]
                  acc = val if acc is None else jnp.maximum(acc, val)
              ebuf[8 + c * 4 + i * 2 + j, pl.ds(v * 16, 16)] = acc
        for v in range(_BN // 16):
          acc = None
          for q in range(4):
            val = ebuf[8 + c * 4 + q, pl.ds(v * 16, 16)]
            acc = val if acc is None else jnp.maximum(acc, val)
          ebuf[c, pl.ds(v * 16, 16)] = acc

      pltpu.sync_copy(ebuf, emb_hbm.at[:, pl.ds(n0, _BN)])
      return carry

    lax.fori_loop(0, blocks_per_w, block_body, 0, unroll=False)

  return ssp


def _router_body(thr_ref, emb_ref, keys_ref, out_ref):
  emb = emb_ref[...]
  s = jnp.sum(emb * emb, axis=0, keepdims=True)
  emb = emb / jnp.maximum(jnp.sqrt(s), 1e-12)
  logits = lax.dot_general(
      keys_ref[...], emb, (((1,), (0,)), ((), ())),
      preferred_element_type=jnp.float32)
  m = jnp.max(logits, axis=0, keepdims=True)
  e = jnp.exp(logits - m)
  w = e / jnp.sum(e, axis=0, keepdims=True)
  t = thr_ref[0]
  wf = jnp.where(w > t, w, 0.0)
  out_ref[...] = wf / (jnp.sum(wf, axis=0, keepdims=True) + 1e-8)


_N_CHUNKS = 4
_TC_BN = 2048


def kernel(patch, keys, threshold):
  n = patch.shape[0]
  pt = jnp.transpose(patch, (1, 2, 3, 0)).reshape(_ROWS, n)
  thr = jnp.reshape(threshold, (1,))
  n_chunk = n // _N_CHUNKS

  def dense(emb):
    bn = min(_TC_BN, n_chunk)
    return pl.pallas_call(
        _router_body,
        grid=(n_chunk // bn,),
        in_specs=[
            pl.BlockSpec(memory_space=pltpu.SMEM),
            pl.BlockSpec((_ED, bn), lambda i: (0, i)),
            pl.BlockSpec((64, _ED), lambda i: (0, 0)),
        ],
        out_specs=pl.BlockSpec((64, bn), lambda i: (0, i)),
        out_shape=jax.ShapeDtypeStruct((64, n_chunk), jnp.float32),
    )(thr, emb, keys)

  outs = []
  for k in range(_N_CHUNKS):
    emb_k = _ssp_sc_kernel(n, k, n_chunk)(pt)
    outs.append(dense(emb_k))
  out_t = jnp.concatenate(outs, axis=1) if _N_CHUNKS > 1 else outs[0]
  return jnp.transpose(out_t)


# speedup vs baseline: 1.1032x; 1.1032x over previous
import functools

import jax
import jax.numpy as jnp
from jax import lax
from jax.experimental import pallas as pl
from jax.experimental.pallas import tpu as pltpu
from jax.experimental.pallas import tpu_sc as plsc

_NC = 2
_NS = 16
_NW = _NC * _NS
_BN = 128
_C = 8
_ROWS = 2048
_ED = 168


def _ssp_sc_kernel(n: int, chunk: int, n_chunk: int):
  blocks_per_w = n_chunk // _BN // _NW
  col0 = chunk * n_chunk
  mesh = plsc.VectorSubcoreMesh(core_axis_name="c", subcore_axis_name="s")

  @functools.partial(
      pl.kernel,
      out_type=jax.ShapeDtypeStruct((_ED, n_chunk), jnp.float32),
      mesh=mesh,
      scratch_types=[
          pltpu.VMEM((256, _BN), jnp.float32),
          pltpu.VMEM((256, _BN), jnp.float32),
          pltpu.VMEM((_ED, _BN), jnp.float32),
          pltpu.SemaphoreType.DMA,
          pltpu.SemaphoreType.DMA,
      ],
      compiler_params=pltpu.CompilerParams(needs_layout_passes=False),
  )
  def ssp(pt_hbm, emb_hbm, buf0, buf1, ebuf, sem0, sem1):
    wid = lax.axis_index("s") * _NC + lax.axis_index("c")
    bufs = (buf0, buf1)
    sems = (sem0, sem1)

    def block_body(k, carry):
      n0 = (wid * blocks_per_w + k) * _BN

      def start(c, slot):
        return pltpu.async_copy(
            pt_hbm.at[pl.ds(c * 256, 256), pl.ds(col0 + n0, _BN)], bufs[slot],
            sems[slot])

      desc = start(0, 0)
      for c in range(_C):
        desc.wait()
        if c + 1 < _C:
          desc = start(c + 1, (c + 1) % 2)
        buf = bufs[c % 2]

        def gj_body(gj, carry2):
          g = gj // 4
          j4 = gj % 4
          accs = []
          for v in range(_BN // 16):
            acc = None
            for dy in range(4):
              for dx in range(4):
                val = buf[(g * 4 + dy) * 16 + j4 * 4 + dx, pl.ds(v * 16, 16)]
                acc = val if acc is None else jnp.maximum(acc, val)
            accs.append(acc)
          for v in range(_BN // 16):
            ebuf[40 + c * 16 + gj, pl.ds(v * 16, 16)] = accs[v]
          return carry2

        lax.fori_loop(0, 16, gj_body, 0, unroll=False)

        for i in range(2):
          for j in range(2):
            for v in range(_BN // 16):
              acc = None
              for di in range(2):
                for dj in range(2):
                  val = ebuf[40 + c * 16 + (2 * i + di) * 4 + (2 * j + dj),
                             pl.ds(v * 16, 16)]
                  acc = val if acc is None else jnp.maximum(acc, val)
              ebuf[8 + c * 4 + i * 2 + j, pl.ds(v * 16, 16)] = acc
        for v in range(_BN // 16):
          acc = None
          for q in range(4):
            val = ebuf[8 + c * 4 + q, pl.ds(v * 16, 16)]
            acc = val if acc is None else jnp.maximum(acc, val)
          ebuf[c, pl.ds(v * 16, 16)] = acc

      pltpu.sync_copy(ebuf, emb_hbm.at[:, pl.ds(n0, _BN)])
      return carry

    lax.fori_loop(0, blocks_per_w, block_body, 0, unroll=False)

  return ssp


def _router_body(thr_ref, emb_ref, keys_ref, out_ref):
  emb = emb_ref[...]
  s = jnp.sum(emb * emb, axis=0, keepdims=True)
  emb = emb / jnp.maximum(jnp.sqrt(s), 1e-12)
  logits = lax.dot_general(
      keys_ref[...], emb, (((1,), (0,)), ((), ())),
      preferred_element_type=jnp.float32)
  m = jnp.max(logits, axis=0, keepdims=True)
  e = jnp.exp(logits - m)
  w = e / jnp.sum(e, axis=0, keepdims=True)
  t = thr_ref[0]
  wf = jnp.where(w > t, w, 0.0)
  out_ref[...] = wf / (jnp.sum(wf, axis=0, keepdims=True) + 1e-8)


_N_CHUNKS = 2
_TC_BN = 2048


def kernel(patch, keys, threshold):
  n = patch.shape[0]
  pt = jnp.transpose(patch, (1, 2, 3, 0)).reshape(_ROWS, n)
  thr = jnp.reshape(threshold, (1,))
  n_chunk = n // _N_CHUNKS

  def dense(emb):
    bn = min(_TC_BN, n_chunk)
    return pl.pallas_call(
        _router_body,
        grid=(n_chunk // bn,),
        in_specs=[
            pl.BlockSpec(memory_space=pltpu.SMEM),
            pl.BlockSpec((_ED, bn), lambda i: (0, i)),
            pl.BlockSpec((64, _ED), lambda i: (0, 0)),
        ],
        out_specs=pl.BlockSpec((64, bn), lambda i: (0, i)),
        out_shape=jax.ShapeDtypeStruct((64, n_chunk), jnp.float32),
    )(thr, emb, keys)

  outs = []
  for k in range(_N_CHUNKS):
    emb_k = _ssp_sc_kernel(n, k, n_chunk)(pt)
    outs.append(dense(emb_k))
  out_t = jnp.concatenate(outs, axis=1) if _N_CHUNKS > 1 else outs[0]
  return jnp.transpose(out_t)
